# baseline (speedup 1.0000x reference)
import functools

import jax
import jax.numpy as jnp
from jax import lax
from jax.experimental import pallas as pl
from jax.experimental.pallas import tpu as pltpu
from jax.experimental.pallas import tpu_sc as plsc

_MASKING_RATIO = 0.75


def _lex_lt(av, ai, bv, bi):
    return (av < bv) | ((av == bv) & (ai < bi))


def _substage(val, key_i, pos, j, k):
    low = (pos & j) == 0
    pval = jnp.where(low, jnp.roll(val, -j, axis=1), jnp.roll(val, j, axis=1))
    pidx = jnp.where(low, jnp.roll(key_i, -j, axis=1),
                     jnp.roll(key_i, j, axis=1))
    lt = _lex_lt(val, key_i, pval, pidx)
    up = (pos & k) == 0
    take_self = lt == (low == up)
    return jnp.where(take_self, val, pval), jnp.where(take_self, key_i, pidx)


def _sort_mask_body(noise_ref, gids_ref, mask_ref):
    n, l = noise_ref.shape
    keep = gids_ref.shape[1]
    orig = noise_ref[...]
    pos = lax.broadcasted_iota(jnp.int32, (n, l), 1)
    val = orig
    key_i = pos
    k = 2
    while k < l:
        j = k // 2
        while j >= 1:
            val, key_i = _substage(val, key_i, pos, j, k)
            j //= 2
        k *= 2
    j = l // 2
    width = l
    while width > keep:
        half = width // 2
        lo_v, hi_v = val[:, :half], val[:, half:]
        lo_i, hi_i = key_i[:, :half], key_i[:, half:]
        swap = _lex_lt(hi_v, hi_i, lo_v, lo_i)
        val = jnp.where(swap, hi_v, lo_v)
        key_i = jnp.where(swap, hi_i, lo_i)
        if half == keep:
            disc_v = jnp.where(swap, lo_v, hi_v)
            disc_i = jnp.where(swap, lo_i, hi_i)
        width = half
    pos_k = lax.broadcasted_iota(jnp.int32, (n, keep), 1)
    j = keep // 2
    while j >= 1:
        val, key_i = _substage(val, key_i, pos_k, j, 2 * keep)
        j //= 2
    w = keep
    while w > 1:
        h = w // 2
        a_v, b_v = disc_v[:, :h], disc_v[:, h:w]
        a_i, b_i = disc_i[:, :h], disc_i[:, h:w]
        t = _lex_lt(b_v, b_i, a_v, a_i)
        disc_v = jnp.where(t, b_v, a_v)
        disc_i = jnp.where(t, b_i, a_i)
        w = h
    row = lax.broadcasted_iota(jnp.int32, (n, keep), 0) + pl.program_id(0) * n
    gids_ref[...] = key_i + row * l
    tv = disc_v[:, :1]
    ti = disc_i[:, :1]
    mask = (orig > tv) | ((orig == tv) & (pos >= ti))
    mask_ref[...] = mask.astype(mask_ref.dtype)


def _sort_mask(noise, keep, rows_per_step=16):
    n, l = noise.shape
    steps = n // rows_per_step
    return pl.pallas_call(
        _sort_mask_body,
        grid=(steps,),
        in_specs=[pl.BlockSpec((rows_per_step, l), lambda i: (i, 0))],
        out_specs=(
            pl.BlockSpec((rows_per_step, keep), lambda i: (i, 0)),
            pl.BlockSpec((rows_per_step, l), lambda i: (i, 0)),
        ),
        out_shape=(
            jax.ShapeDtypeStruct((n, keep), jnp.int32),
            jax.ShapeDtypeStruct((n, l), jnp.float32),
        ),
    )(noise)


def _make_sc_gather(n, keep, d, chunk):
    rows = n * keep
    info = plsc.get_sparse_core_info()
    nc, ns = info.num_cores, info.num_subcores
    nw = nc * ns
    per_w = rows // nw
    rows_per_w = per_w // keep
    assert per_w % chunk == 0 and per_w % 8 == 0 and chunk % 8 == 0
    assert per_w % keep == 0 and keep % chunk == 0
    n_chunks = per_w // chunk
    mesh = plsc.VectorSubcoreMesh(core_axis_name="c", subcore_axis_name="s")

    @functools.partial(
        pl.kernel,
        mesh=mesh,
        out_type=jax.ShapeDtypeStruct((rows, d), jnp.float32),
        scratch_types=(
            [pltpu.VMEM((per_w,), jnp.int32)]
            + [pltpu.VMEM((chunk, d), jnp.float32) for _ in range(4)]
            + [pltpu.SemaphoreType.DMA for _ in range(8)]
        ),
    )
    def gather(table_hbm, gids_hbm, out_hbm, idx_all, *bufsem):
        nbuf = 4
        bufs = bufsem[:nbuf]
        gsem = bufsem[nbuf:2 * nbuf]
        wsem = bufsem[2 * nbuf:3 * nbuf]
        wid = lax.axis_index("s") * nc + lax.axis_index("c")
        base = wid * per_w
        for r in range(rows_per_w):
            pltpu.sync_copy(gids_hbm.at[wid * rows_per_w + r],
                            idx_all.at[pl.ds(r * keep, keep)])

        def start(c):
            return pltpu.async_copy(
                table_hbm.at[idx_all.at[pl.ds(c * chunk, chunk)]],
                bufs[c % nbuf], gsem[c % nbuf])

        def wback(c):
            return pltpu.async_copy(
                bufs[c % nbuf], out_hbm.at[pl.ds(base + c * chunk, chunk)],
                wsem[c % nbuf])

        gcp = [None] * n_chunks
        wcp = [None] * n_chunks
        for c in range(min(nbuf, n_chunks)):
            gcp[c] = start(c)
        for c in range(n_chunks):
            gcp[c].wait()
            wcp[c] = wback(c)
            if c + nbuf < n_chunks:
                wcp[c].wait()
                gcp[c + nbuf] = start(c + nbuf)
        for c in range(max(0, n_chunks - nbuf), n_chunks):
            wcp[c].wait()

    return gather


def kernel(x, noise):
    n, l, d = x.shape
    keep = int(l * (1 - _MASKING_RATIO))
    gids, mask = _sort_mask(noise, keep)
    table = x.reshape(n * l, d)
    gather = _make_sc_gather(n, keep, d, chunk=32)
    x_masked = gather(table, gids)
    return x_masked.reshape(n, keep, d), mask


# speedup vs baseline: 1.0839x; 1.0839x over previous
import functools

import jax
import jax.numpy as jnp
from jax import lax
from jax.experimental import pallas as pl
from jax.experimental.pallas import tpu as pltpu
from jax.experimental.pallas import tpu_sc as plsc

_MASKING_RATIO = 0.75


def _lex_lt(av, ai, bv, bi):
    return (av < bv) | ((av == bv) & (ai < bi))


def _substage(val, key_i, pos, j, k):
    low = (pos & j) == 0
    pval = jnp.where(low, jnp.roll(val, -j, axis=1), jnp.roll(val, j, axis=1))
    pidx = jnp.where(low, jnp.roll(key_i, -j, axis=1),
                     jnp.roll(key_i, j, axis=1))
    lt = _lex_lt(val, key_i, pval, pidx)
    up = (pos & k) == 0
    take_self = lt == (low == up)
    return jnp.where(take_self, val, pval), jnp.where(take_self, key_i, pidx)


def _sort_mask_body(noise_ref, gids_ref, mask_ref):
    n, l = noise_ref.shape
    keep = gids_ref.shape[1]
    orig = noise_ref[...]
    pos = lax.broadcasted_iota(jnp.int32, (n, l), 1)
    val = orig
    key_i = pos
    k = 2
    while k < l:
        j = k // 2
        while j >= 1:
            val, key_i = _substage(val, key_i, pos, j, k)
            j //= 2
        k *= 2
    j = l // 2
    width = l
    while width > keep:
        half = width // 2
        lo_v, hi_v = val[:, :half], val[:, half:]
        lo_i, hi_i = key_i[:, :half], key_i[:, half:]
        swap = _lex_lt(hi_v, hi_i, lo_v, lo_i)
        val = jnp.where(swap, hi_v, lo_v)
        key_i = jnp.where(swap, hi_i, lo_i)
        if half == keep:
            disc_v = jnp.where(swap, lo_v, hi_v)
            disc_i = jnp.where(swap, lo_i, hi_i)
        width = half
    pos_k = lax.broadcasted_iota(jnp.int32, (n, keep), 1)
    j = keep // 2
    while j >= 1:
        val, key_i = _substage(val, key_i, pos_k, j, 2 * keep)
        j //= 2
    w = keep
    while w > 1:
        h = w // 2
        a_v, b_v = disc_v[:, :h], disc_v[:, h:w]
        a_i, b_i = disc_i[:, :h], disc_i[:, h:w]
        t = _lex_lt(b_v, b_i, a_v, a_i)
        disc_v = jnp.where(t, b_v, a_v)
        disc_i = jnp.where(t, b_i, a_i)
        w = h
    row = lax.broadcasted_iota(jnp.int32, (n, keep), 0) + pl.program_id(0) * n
    gids_ref[...] = key_i + row * l
    tv = disc_v[:, :1]
    ti = disc_i[:, :1]
    mask = (orig > tv) | ((orig == tv) & (pos >= ti))
    mask_ref[...] = mask.astype(mask_ref.dtype)


def _sort_mask(noise, keep, rows_per_step=32):
    n, l = noise.shape
    steps = n // rows_per_step
    return pl.pallas_call(
        _sort_mask_body,
        grid=(steps,),
        in_specs=[pl.BlockSpec((rows_per_step, l), lambda i: (i, 0))],
        out_specs=(
            pl.BlockSpec((rows_per_step, keep), lambda i: (i, 0)),
            pl.BlockSpec((rows_per_step, l), lambda i: (i, 0)),
        ),
        out_shape=(
            jax.ShapeDtypeStruct((n, keep), jnp.int32),
            jax.ShapeDtypeStruct((n, l), jnp.float32),
        ),
    )(noise)


def _make_sc_gather(n, keep, d, chunk):
    rows = n * keep
    info = plsc.get_sparse_core_info()
    nc, ns = info.num_cores, info.num_subcores
    nw = nc * ns
    per_w = rows // nw
    rows_per_w = per_w // keep
    assert per_w % chunk == 0 and per_w % 8 == 0 and chunk % 8 == 0
    assert per_w % keep == 0 and keep % chunk == 0
    n_chunks = per_w // chunk
    mesh = plsc.VectorSubcoreMesh(core_axis_name="c", subcore_axis_name="s")

    @functools.partial(
        pl.kernel,
        mesh=mesh,
        out_type=jax.ShapeDtypeStruct((rows, d), jnp.float32),
        scratch_types=(
            [pltpu.VMEM((per_w,), jnp.int32)]
            + [pltpu.VMEM((chunk, d), jnp.float32) for _ in range(4)]
            + [pltpu.SemaphoreType.DMA for _ in range(8)]
        ),
    )
    def gather(table_hbm, gids_hbm, out_hbm, idx_all, *bufsem):
        nbuf = 4
        bufs = bufsem[:nbuf]
        gsem = bufsem[nbuf:2 * nbuf]
        wsem = bufsem[2 * nbuf:3 * nbuf]
        wid = lax.axis_index("s") * nc + lax.axis_index("c")
        base = wid * per_w
        for r in range(rows_per_w):
            pltpu.sync_copy(gids_hbm.at[wid * rows_per_w + r],
                            idx_all.at[pl.ds(r * keep, keep)])

        def start(c):
            return pltpu.async_copy(
                table_hbm.at[idx_all.at[pl.ds(c * chunk, chunk)]],
                bufs[c % nbuf], gsem[c % nbuf])

        def wback(c):
            return pltpu.async_copy(
                bufs[c % nbuf], out_hbm.at[pl.ds(base + c * chunk, chunk)],
                wsem[c % nbuf])

        gcp = [None] * n_chunks
        wcp = [None] * n_chunks
        for c in range(min(nbuf, n_chunks)):
            gcp[c] = start(c)
        for c in range(n_chunks):
            gcp[c].wait()
            wcp[c] = wback(c)
            if c + nbuf < n_chunks:
                wcp[c].wait()
                gcp[c + nbuf] = start(c + nbuf)
        for c in range(max(0, n_chunks - nbuf), n_chunks):
            wcp[c].wait()

    return gather


def kernel(x, noise):
    n, l, d = x.shape
    keep = int(l * (1 - _MASKING_RATIO))
    gids, mask = _sort_mask(noise, keep)
    table = x.reshape(n * l, d)
    gather = _make_sc_gather(n, keep, d, chunk=32)
    x_masked = gather(table, gids)
    return x_masked.reshape(n, keep, d), mask


# speedup vs baseline: 1.1422x; 1.0538x over previous
import functools

import jax
import jax.numpy as jnp
from jax import lax
from jax.experimental import pallas as pl
from jax.experimental.pallas import tpu as pltpu
from jax.experimental.pallas import tpu_sc as plsc

_MASKING_RATIO = 0.75


def _lex_lt(av, ai, bv, bi):
    return (av < bv) | ((av == bv) & (ai < bi))


def _substage(val, key_i, pos, j, k):
    low = (pos & j) == 0
    pval = jnp.where(low, jnp.roll(val, -j, axis=1), jnp.roll(val, j, axis=1))
    pidx = jnp.where(low, jnp.roll(key_i, -j, axis=1),
                     jnp.roll(key_i, j, axis=1))
    lt = _lex_lt(val, key_i, pval, pidx)
    up = (pos & k) == 0
    take_self = lt == (low == up)
    return jnp.where(take_self, val, pval), jnp.where(take_self, key_i, pidx)


def _sort_mask_body(noise_ref, gids_ref, mask_ref):
    n, l = noise_ref.shape
    keep = gids_ref.shape[1]
    orig = noise_ref[...]
    pos = lax.broadcasted_iota(jnp.int32, (n, l), 1)
    val = orig
    key_i = pos
    k = 2
    while k <= keep:
        j = k // 2
        while j >= 1:
            val, key_i = _substage(val, key_i, pos, j, k)
            j //= 2
        k *= 2
    disc_vs, disc_is = [], []
    width = l
    while width > keep:
        half = width // 2
        nb = half // keep
        lo_v = jnp.concatenate(
            [val[:, 2 * b * keep:(2 * b + 1) * keep] for b in range(nb)], axis=1)
        hi_v = jnp.concatenate(
            [val[:, (2 * b + 1) * keep:(2 * b + 2) * keep] for b in range(nb)], axis=1)
        lo_i = jnp.concatenate(
            [key_i[:, 2 * b * keep:(2 * b + 1) * keep] for b in range(nb)], axis=1)
        hi_i = jnp.concatenate(
            [key_i[:, (2 * b + 1) * keep:(2 * b + 2) * keep] for b in range(nb)], axis=1)
        swap = _lex_lt(hi_v, hi_i, lo_v, lo_i)
        val = jnp.where(swap, hi_v, lo_v)
        key_i = jnp.where(swap, hi_i, lo_i)
        disc_vs.append(jnp.where(swap, lo_v, hi_v))
        disc_is.append(jnp.where(swap, lo_i, hi_i))
        width = half
        if width > keep:
            pos_w = lax.broadcasted_iota(jnp.int32, (n, width), 1)
            j = keep // 2
            while j >= 1:
                val, key_i = _substage(val, key_i, pos_w, j, keep)
                j //= 2
    pos_k = lax.broadcasted_iota(jnp.int32, (n, keep), 1)
    j = keep // 2
    while j >= 1:
        val, key_i = _substage(val, key_i, pos_k, j, 2 * keep)
        j //= 2
    disc_v, disc_i = None, None
    for dv, di in zip(disc_vs, disc_is):
        w = dv.shape[1]
        while w > 1:
            h = w // 2
            a_v, b_v = dv[:, :h], dv[:, h:w]
            a_i, b_i = di[:, :h], di[:, h:w]
            t = _lex_lt(b_v, b_i, a_v, a_i)
            dv = jnp.where(t, b_v, a_v)
            di = jnp.where(t, b_i, a_i)
            w = h
        if disc_v is None:
            disc_v, disc_i = dv, di
        else:
            t = _lex_lt(dv, di, disc_v, disc_i)
            disc_v = jnp.where(t, dv, disc_v)
            disc_i = jnp.where(t, di, disc_i)
    row = lax.broadcasted_iota(jnp.int32, (n, keep), 0)
    gids_ref[...] = key_i + row * l
    tv = disc_v[:, :1]
    ti = disc_i[:, :1]
    mask = (orig > tv) | ((orig == tv) & (pos >= ti))
    mask_ref[...] = mask.astype(mask_ref.dtype)


def _sort_mask(noise, keep):
    n, l = noise.shape
    return pl.pallas_call(
        _sort_mask_body,
        out_shape=(
            jax.ShapeDtypeStruct((n, keep), jnp.int32),
            jax.ShapeDtypeStruct((n, l), jnp.float32),
        ),
    )(noise)


def _make_sc_gather(n, keep, d, chunk):
    rows = n * keep
    info = plsc.get_sparse_core_info()
    nc, ns = info.num_cores, info.num_subcores
    nw = nc * ns
    per_w = rows // nw
    rows_per_w = per_w // keep
    assert per_w % chunk == 0 and per_w % 8 == 0 and chunk % 8 == 0
    assert per_w % keep == 0 and keep % chunk == 0
    n_chunks = per_w // chunk
    mesh = plsc.VectorSubcoreMesh(core_axis_name="c", subcore_axis_name="s")

    @functools.partial(
        pl.kernel,
        mesh=mesh,
        out_type=jax.ShapeDtypeStruct((rows, d), jnp.float32),
        scratch_types=(
            [pltpu.VMEM((per_w,), jnp.int32)]
            + [pltpu.VMEM((chunk, d), jnp.float32) for _ in range(4)]
            + [pltpu.SemaphoreType.DMA for _ in range(8)]
        ),
    )
    def gather(table_hbm, gids_hbm, out_hbm, idx_all, *bufsem):
        nbuf = 4
        bufs = bufsem[:nbuf]
        gsem = bufsem[nbuf:2 * nbuf]
        wsem = bufsem[2 * nbuf:3 * nbuf]
        wid = lax.axis_index("s") * nc + lax.axis_index("c")
        base = wid * per_w
        for r in range(rows_per_w):
            pltpu.sync_copy(gids_hbm.at[wid * rows_per_w + r],
                            idx_all.at[pl.ds(r * keep, keep)])

        def start(c):
            return pltpu.async_copy(
                table_hbm.at[idx_all.at[pl.ds(c * chunk, chunk)]],
                bufs[c % nbuf], gsem[c % nbuf])

        def wback(c):
            return pltpu.async_copy(
                bufs[c % nbuf], out_hbm.at[pl.ds(base + c * chunk, chunk)],
                wsem[c % nbuf])

        gcp = [None] * n_chunks
        wcp = [None] * n_chunks
        for c in range(min(nbuf, n_chunks)):
            gcp[c] = start(c)
        for c in range(n_chunks):
            gcp[c].wait()
            wcp[c] = wback(c)
            if c + nbuf < n_chunks:
                wcp[c].wait()
                gcp[c + nbuf] = start(c + nbuf)
        for c in range(max(0, n_chunks - nbuf), n_chunks):
            wcp[c].wait()

    return gather


def kernel(x, noise):
    n, l, d = x.shape
    keep = int(l * (1 - _MASKING_RATIO))
    gids, mask = _sort_mask(noise, keep)
    table = x.reshape(n * l, d)
    gather = _make_sc_gather(n, keep, d, chunk=32)
    x_masked = gather(table, gids)
    return x_masked.reshape(n, keep, d), mask


# speedup vs baseline: 1.1624x; 1.0176x over previous
import functools

import jax
import jax.numpy as jnp
from jax import lax
from jax.experimental import pallas as pl
from jax.experimental.pallas import tpu as pltpu
from jax.experimental.pallas import tpu_sc as plsc

_MASKING_RATIO = 0.75


def _lex_lt(av, ai, bv, bi):
    return (av < bv) | ((av == bv) & (ai < bi))


def _substage(val, key_i, pos, j, k):
    low = (pos & j) == 0
    pval = jnp.where(low, jnp.roll(val, -j, axis=1), jnp.roll(val, j, axis=1))
    pidx = jnp.where(low, jnp.roll(key_i, -j, axis=1),
                     jnp.roll(key_i, j, axis=1))
    lt = _lex_lt(val, key_i, pval, pidx)
    up = (pos & k) == 0
    take_self = lt == (low == up)
    return jnp.where(take_self, val, pval), jnp.where(take_self, key_i, pidx)


def _sort_mask_body(noise_ref, gids_ref, mask_ref):
    n, l = noise_ref.shape
    keep = gids_ref.shape[1]
    orig = noise_ref[...]
    pos = lax.broadcasted_iota(jnp.int32, (n, l), 1)
    val = orig
    key_i = pos
    k = 2
    while k <= keep:
        j = k // 2
        while j >= 1:
            val, key_i = _substage(val, key_i, pos, j, k)
            j //= 2
        k *= 2
    disc_vs, disc_is = [], []
    width = l
    while width > keep:
        half = width // 2
        nb = half // keep
        lo_v = jnp.concatenate(
            [val[:, 2 * b * keep:(2 * b + 1) * keep] for b in range(nb)], axis=1)
        hi_v = jnp.concatenate(
            [val[:, (2 * b + 1) * keep:(2 * b + 2) * keep] for b in range(nb)], axis=1)
        lo_i = jnp.concatenate(
            [key_i[:, 2 * b * keep:(2 * b + 1) * keep] for b in range(nb)], axis=1)
        hi_i = jnp.concatenate(
            [key_i[:, (2 * b + 1) * keep:(2 * b + 2) * keep] for b in range(nb)], axis=1)
        swap = _lex_lt(hi_v, hi_i, lo_v, lo_i)
        val = jnp.where(swap, hi_v, lo_v)
        key_i = jnp.where(swap, hi_i, lo_i)
        disc_vs.append(jnp.where(swap, lo_v, hi_v))
        disc_is.append(jnp.where(swap, lo_i, hi_i))
        width = half
        if width > keep:
            pos_w = lax.broadcasted_iota(jnp.int32, (n, width), 1)
            j = keep // 2
            while j >= 1:
                val, key_i = _substage(val, key_i, pos_w, j, keep)
                j //= 2
    pos_k = lax.broadcasted_iota(jnp.int32, (n, keep), 1)
    j = keep // 2
    while j >= 1:
        val, key_i = _substage(val, key_i, pos_k, j, 2 * keep)
        j //= 2
    disc_v, disc_i = None, None
    for dv, di in zip(disc_vs, disc_is):
        w = dv.shape[1]
        while w > 1:
            h = w // 2
            a_v, b_v = dv[:, :h], dv[:, h:w]
            a_i, b_i = di[:, :h], di[:, h:w]
            t = _lex_lt(b_v, b_i, a_v, a_i)
            dv = jnp.where(t, b_v, a_v)
            di = jnp.where(t, b_i, a_i)
            w = h
        if disc_v is None:
            disc_v, disc_i = dv, di
        else:
            t = _lex_lt(dv, di, disc_v, disc_i)
            disc_v = jnp.where(t, dv, disc_v)
            disc_i = jnp.where(t, di, disc_i)
    row = lax.broadcasted_iota(jnp.int32, (n, keep), 0)
    gids_ref[...] = key_i + row * l
    tv = disc_v[:, :1]
    ti = disc_i[:, :1]
    mask = (orig > tv) | ((orig == tv) & (pos >= ti))
    mask_ref[...] = mask.astype(mask_ref.dtype)


def _sort_mask(noise, keep):
    n, l = noise.shape
    return pl.pallas_call(
        _sort_mask_body,
        out_shape=(
            jax.ShapeDtypeStruct((n, keep), jnp.int32),
            jax.ShapeDtypeStruct((n, l), jnp.float32),
        ),
    )(noise)


def _make_sc_gather(n, keep, d, chunk):
    rows = n * keep
    info = plsc.get_sparse_core_info()
    nc, ns = info.num_cores, info.num_subcores
    nw = nc * ns
    per_w = rows // nw
    rows_per_w = per_w // keep
    assert per_w % chunk == 0 and per_w % 8 == 0 and chunk % 8 == 0
    assert per_w % keep == 0 and keep % chunk == 0
    n_chunks = per_w // chunk
    mesh = plsc.VectorSubcoreMesh(core_axis_name="c", subcore_axis_name="s")

    @functools.partial(
        pl.kernel,
        mesh=mesh,
        out_type=jax.ShapeDtypeStruct((rows, d), jnp.float32),
        scratch_types=(
            [pltpu.VMEM((per_w,), jnp.int32)]
            + [pltpu.VMEM((chunk, d), jnp.float32) for _ in range(5)]
            + [pltpu.SemaphoreType.DMA for _ in range(10)]
        ),
    )
    def gather(table_hbm, gids_hbm, out_hbm, idx_all, *bufsem):
        nbuf = 5
        bufs = bufsem[:nbuf]
        gsem = bufsem[nbuf:2 * nbuf]
        wsem = bufsem[2 * nbuf:3 * nbuf]
        wid = lax.axis_index("s") * nc + lax.axis_index("c")
        base = wid * per_w

        def load_ids(r):
            pltpu.sync_copy(gids_hbm.at[wid * rows_per_w + r],
                            idx_all.at[pl.ds(r * keep, keep)])

        load_ids(0)

        def start(c):
            return pltpu.async_copy(
                table_hbm.at[idx_all.at[pl.ds(c * chunk, chunk)]],
                bufs[c % nbuf], gsem[c % nbuf])

        def wback(c):
            return pltpu.async_copy(
                bufs[c % nbuf], out_hbm.at[pl.ds(base + c * chunk, chunk)],
                wsem[c % nbuf])

        gcp = [None] * n_chunks
        wcp = [None] * n_chunks
        for c in range(min(nbuf, n_chunks)):
            gcp[c] = start(c)
        for r in range(1, rows_per_w):
            load_ids(r)
        for c in range(n_chunks):
            gcp[c].wait()
            wcp[c] = wback(c)
            if c + nbuf < n_chunks:
                wcp[c].wait()
                gcp[c + nbuf] = start(c + nbuf)
        for c in range(max(0, n_chunks - nbuf), n_chunks):
            wcp[c].wait()

    return gather


def kernel(x, noise):
    n, l, d = x.shape
    keep = int(l * (1 - _MASKING_RATIO))
    gids, mask = _sort_mask(noise, keep)
    table = x.reshape(n * l, d)
    gather = _make_sc_gather(n, keep, d, chunk=32)
    x_masked = gather(table, gids)
    return x_masked.reshape(n, keep, d), mask
